# Initial kernel scaffold; baseline (speedup 1.0000x reference)
import jax
import jax.numpy as jnp
from jax.experimental import pallas as pl


def kernel(inputs):
    raise NotImplementedError("write your pallas kernel here")


# speedup vs baseline: 25.6493x; 25.6493x over previous
import jax
import jax.numpy as jnp
from jax.experimental import pallas as pl

_K = 8


def _topk_kernel(x_ref, o_ref):
    x = x_ref[0]
    t = x.shape[0]
    row = jax.lax.broadcasted_iota(jnp.int32, x.shape, 0)
    for k in range(_K):
        m = jnp.max(x, axis=0, keepdims=True)
        o_ref[0, k, :] = m[0]
        if k < _K - 1:
            cand = jnp.where(x == m, row, t)
            first = jnp.min(cand, axis=0, keepdims=True)
            x = jnp.where(row == first, -jnp.inf, x)


def kernel(inputs):
    b, t, c = inputs.shape
    cb = 256
    out = pl.pallas_call(
        _topk_kernel,
        grid=(b, c // cb),
        in_specs=[pl.BlockSpec((1, t, cb), lambda i, j: (i, 0, j))],
        out_specs=pl.BlockSpec((1, _K, cb), lambda i, j: (i, 0, j)),
        out_shape=jax.ShapeDtypeStruct((b, _K, c), inputs.dtype),
    )(inputs)
    return out.reshape(b, _K * c)


# speedup vs baseline: 72.0674x; 2.8097x over previous
import jax
import jax.numpy as jnp
from jax.experimental import pallas as pl

_K = 8

_SORT8 = [
    (0, 1), (2, 3), (4, 5), (6, 7),
    (0, 2), (1, 3), (4, 6), (5, 7),
    (1, 2), (5, 6),
    (0, 4), (1, 5), (2, 6), (3, 7),
    (2, 4), (3, 5),
    (1, 2), (3, 4), (5, 6),
]

_BITONIC8 = [
    (0, 4), (1, 5), (2, 6), (3, 7),
    (0, 2), (1, 3), (4, 6), (5, 7),
    (0, 1), (2, 3), (4, 5), (6, 7),
]


def _cx(a, i, j):
    hi = jnp.maximum(a[i], a[j])
    lo = jnp.minimum(a[i], a[j])
    a[i] = hi
    a[j] = lo


def _topk_kernel(x_ref, o_ref):
    x = x_ref[0]
    t = x.shape[0]
    g = t // _K
    a = [x[i * g:(i + 1) * g, :] for i in range(_K)]
    for (i, j) in _SORT8:
        _cx(a, i, j)
    while g > 1:
        h = g // 2
        top = [v[:h, :] for v in a]
        bot = [v[h:, :] for v in a]
        a = [jnp.maximum(top[i], bot[_K - 1 - i]) for i in range(_K)]
        for (i, j) in _BITONIC8:
            _cx(a, i, j)
        g = h
    for i in range(_K):
        o_ref[0, i, :] = a[i][0]


def kernel(inputs):
    b, t, c = inputs.shape
    cb = 256
    out = pl.pallas_call(
        _topk_kernel,
        grid=(b, c // cb),
        in_specs=[pl.BlockSpec((1, t, cb), lambda i, j: (i, 0, j))],
        out_specs=pl.BlockSpec((1, _K, cb), lambda i, j: (i, 0, j)),
        out_shape=jax.ShapeDtypeStruct((b, _K, c), inputs.dtype),
    )(inputs)
    return out.reshape(b, _K * c)


# speedup vs baseline: 115.5516x; 1.6034x over previous
import jax
import jax.numpy as jnp
from jax.experimental import pallas as pl

_K = 8

_SORT8 = [
    (0, 1), (2, 3), (4, 5), (6, 7),
    (0, 2), (1, 3), (4, 6), (5, 7),
    (1, 2), (5, 6),
    (0, 4), (1, 5), (2, 6), (3, 7),
    (2, 4), (3, 5),
    (1, 2), (3, 4), (5, 6),
]

_BITONIC8 = [
    (0, 4), (1, 5), (2, 6), (3, 7),
    (0, 2), (1, 3), (4, 6), (5, 7),
    (0, 1), (2, 3), (4, 5), (6, 7),
]


def _cx(a, i, j):
    hi = jnp.maximum(a[i], a[j])
    lo = jnp.minimum(a[i], a[j])
    a[i] = hi
    a[j] = lo


def _merge8(acc, s):
    m = [jnp.maximum(acc[i], s[_K - 1 - i]) for i in range(_K)]
    for (i, j) in _BITONIC8:
        _cx(m, i, j)
    return m


def _topk_kernel(x_ref, o_ref):
    t = x_ref.shape[1]
    chunks = t // 64

    def body(m, acc):
        acc = list(acc)
        base = m * 64
        s = [x_ref[0, pl.ds(base + _K * j, _K), :] for j in range(_K)]
        for (i, j) in _SORT8:
            _cx(s, i, j)
        return tuple(_merge8(acc, s))

    init = tuple(
        jnp.full((_K, x_ref.shape[2]), -jnp.inf, dtype=x_ref.dtype)
        for _ in range(_K)
    )
    acc = jax.lax.fori_loop(0, chunks, body, init, unroll=4)
    a = list(acc)
    h = _K // 2
    while h >= 1:
        top = [v[:h, :] for v in a]
        bot = [v[h:2 * h, :] for v in a]
        a = _merge8(top, bot)
        h //= 2
    for i in range(_K):
        o_ref[0, i, :] = a[i][0]


def kernel(inputs):
    b, t, c = inputs.shape
    cb = 256
    out = pl.pallas_call(
        _topk_kernel,
        grid=(b, c // cb),
        in_specs=[pl.BlockSpec((1, t, cb), lambda i, j: (i, 0, j))],
        out_specs=pl.BlockSpec((1, _K, cb), lambda i, j: (i, 0, j)),
        out_shape=jax.ShapeDtypeStruct((b, _K, c), inputs.dtype),
    )(inputs)
    return out.reshape(b, _K * c)
